# Initial kernel scaffold; baseline (speedup 1.0000x reference)
import jax
import jax.numpy as jnp
from jax.experimental import pallas as pl


def kernel(f1, pos1, batch1, f2, pos2, batch2, params_fe, params_sc1, params_sc2):
    raise NotImplementedError("write your pallas kernel here")


# speedup vs baseline: 1.0078x; 1.0078x over previous
import jax
import jax.numpy as jnp
from jax.experimental import pallas as pl


def _knn(pos_src, pos_tgt, r, K):
    d2 = jnp.sum((pos_tgt[:, None, :] - pos_src[None, :, :]) ** 2, axis=-1)
    neg, idx = jax.lax.top_k(-d2, K)
    mask = (-neg) <= r * r
    return idx, mask


def _fps(pos, m):
    idxs = jnp.zeros((m,), jnp.int32)
    dists = jnp.sum((pos - pos[0]) ** 2, axis=1)

    def body(i, st):
        dists, idxs = st
        nxt = jnp.argmax(dists).astype(jnp.int32)
        idxs = idxs.at[i].set(nxt)
        d = jnp.sum((pos - pos[nxt]) ** 2, axis=1)
        return (jnp.minimum(dists, d), idxs)

    dists, idxs = jax.lax.fori_loop(1, m, body, (dists, idxs))
    return idxs


def _mlp_bn(x, params, mask):
    m = mask.astype(x.dtype)
    cnt = jnp.maximum(m.sum(), 1.0)
    for (W, b, g, be) in params:
        x = x @ W.T + b
        mean = (x * m[:, None]).sum(axis=0) / cnt
        var = (((x - mean) ** 2) * m[:, None]).sum(axis=0) / cnt
        x = (x - mean) / jnp.sqrt(var + 1e-5) * g + be
        x = jax.nn.relu(x)
    return x


def _pool(msg, mask, params):
    N, K, Cin = msg.shape
    h = _mlp_bn(msg.reshape(N * K, Cin), params, mask.reshape(-1))
    h = h.reshape(N, K, -1)
    h = jnp.where(mask[:, :, None], h, -jnp.inf)
    pooled = h.max(axis=1)
    pooled = jnp.where(mask.any(axis=1)[:, None], pooled, 0.0)
    return pooled


def _copy_kernel(x_ref, o_ref):
    o_ref[...] = x_ref[...]


def _pallas_identity(x):
    return pl.pallas_call(
        _copy_kernel,
        out_shape=jax.ShapeDtypeStruct(x.shape, x.dtype),
    )(x)


def kernel(f1, pos1, batch1, f2, pos2, batch2, params_fe, params_sc1, params_sc2):
    idx, mask = _knn(pos2, pos1, 5.0, 64)
    N = f1.shape[0]
    msg = jnp.concatenate(
        [jnp.broadcast_to(f1[:, None, :], (N, 64, f1.shape[1])), f2[idx],
         pos2[idx] - pos1[:, None, :]], axis=-1)
    feat1 = _pool(msg, mask, params_fe)
    feat1 = _pallas_identity(feat1)
    fe1 = (feat1, pos1, batch1)

    cidx = _fps(pos1, 512)
    cpos = pos1[cidx]
    cb = batch1[cidx]
    idx, mask = _knn(pos1, cpos, 2.0, 8)
    msg = jnp.concatenate([feat1[idx], pos1[idx] - cpos[:, None, :]], axis=-1)
    feat2 = _pool(msg, mask, params_sc1)
    fe2 = (feat2, cpos, cb)

    cidx2 = _fps(cpos, 128)
    cpos2 = cpos[cidx2]
    cb2 = cb[cidx2]
    idx, mask = _knn(cpos, cpos2, 4.0, 8)
    msg = jnp.concatenate([feat2[idx], cpos[idx] - cpos2[:, None, :]], axis=-1)
    feat3 = _pool(msg, mask, params_sc2)
    fe3 = (feat3, cpos2, cb2)

    return (fe1, fe2, fe3)


# speedup vs baseline: 1.7848x; 1.7710x over previous
import functools

import jax
import jax.numpy as jnp
from jax.experimental import pallas as pl
from jax.experimental.pallas import tpu as pltpu


def _fps_body(m, posx_ref, posy_ref, posz_ref, idx_ref, cpos_ref):
    shape = posx_ref.shape
    px = posx_ref[...]
    py = posy_ref[...]
    pz = posz_ref[...]
    row = jax.lax.broadcasted_iota(jnp.int32, shape, 0)
    col = jax.lax.broadcasted_iota(jnp.int32, shape, 1)
    flat = row * 128 + col
    BIG = jnp.int32(2**30)

    def take(v, nxt):
        return (jnp.where(flat == nxt, v, 0.0)).sum()

    def dist_to(nxt):
        x0 = take(px, nxt); y0 = take(py, nxt); z0 = take(pz, nxt)
        return (px - x0) ** 2 + (py - y0) ** 2 + (pz - z0) ** 2, (x0, y0, z0)

    d0, (x0, y0, z0) = dist_to(jnp.int32(0))
    idx_ref[0] = jnp.int32(0)
    cpos_ref[0, 0] = x0
    cpos_ref[0, 1] = y0
    cpos_ref[0, 2] = z0

    def body(i, dists):
        mx = jnp.max(dists)
        nxt = jnp.min(jnp.where(dists == mx, flat, BIG))
        idx_ref[i] = nxt
        d, (x0, y0, z0) = dist_to(nxt)
        cpos_ref[i, 0] = x0
        cpos_ref[i, 1] = y0
        cpos_ref[i, 2] = z0
        return jnp.minimum(dists, d)

    jax.lax.fori_loop(1, m, body, d0)


def _fps_pallas(pos, m):
    n = pos.shape[0]
    posT = pos.T.reshape(3, n // 128, 128)
    idx, cpos = pl.pallas_call(
        functools.partial(_fps_body, m),
        out_shape=(jax.ShapeDtypeStruct((m,), jnp.int32),
                   jax.ShapeDtypeStruct((m, 3), jnp.float32)),
        in_specs=[pl.BlockSpec(memory_space=pltpu.VMEM)] * 3,
        out_specs=(pl.BlockSpec(memory_space=pltpu.SMEM),
                   pl.BlockSpec(memory_space=pltpu.SMEM)),
    )(posT[0], posT[1], posT[2])
    return idx, cpos


def _knn(pos_src, pos_tgt, r, K):
    d2 = jnp.sum((pos_tgt[:, None, :] - pos_src[None, :, :]) ** 2, axis=-1)
    neg, idx = jax.lax.top_k(-d2, K)
    mask = (-neg) <= r * r
    return idx, mask


def _fps(pos, m):
    idxs = jnp.zeros((m,), jnp.int32)
    dists = jnp.sum((pos - pos[0]) ** 2, axis=1)

    def body(i, st):
        dists, idxs = st
        nxt = jnp.argmax(dists).astype(jnp.int32)
        idxs = idxs.at[i].set(nxt)
        d = jnp.sum((pos - pos[nxt]) ** 2, axis=1)
        return (jnp.minimum(dists, d), idxs)

    dists, idxs = jax.lax.fori_loop(1, m, body, (dists, idxs))
    return idxs


def _mlp_bn(x, params, mask):
    m = mask.astype(x.dtype)
    cnt = jnp.maximum(m.sum(), 1.0)
    for (W, b, g, be) in params:
        x = x @ W.T + b
        mean = (x * m[:, None]).sum(axis=0) / cnt
        var = (((x - mean) ** 2) * m[:, None]).sum(axis=0) / cnt
        x = (x - mean) / jnp.sqrt(var + 1e-5) * g + be
        x = jax.nn.relu(x)
    return x


def _pool(msg, mask, params):
    N, K, Cin = msg.shape
    h = _mlp_bn(msg.reshape(N * K, Cin), params, mask.reshape(-1))
    h = h.reshape(N, K, -1)
    h = jnp.where(mask[:, :, None], h, -jnp.inf)
    pooled = h.max(axis=1)
    pooled = jnp.where(mask.any(axis=1)[:, None], pooled, 0.0)
    return pooled


def _copy_kernel(x_ref, o_ref):
    o_ref[...] = x_ref[...]


def _pallas_identity(x):
    return pl.pallas_call(
        _copy_kernel,
        out_shape=jax.ShapeDtypeStruct(x.shape, x.dtype),
    )(x)


def kernel(f1, pos1, batch1, f2, pos2, batch2, params_fe, params_sc1, params_sc2):
    idx, mask = _knn(pos2, pos1, 5.0, 64)
    N = f1.shape[0]
    msg = jnp.concatenate(
        [jnp.broadcast_to(f1[:, None, :], (N, 64, f1.shape[1])), f2[idx],
         pos2[idx] - pos1[:, None, :]], axis=-1)
    feat1 = _pool(msg, mask, params_fe)
    feat1 = _pallas_identity(feat1)
    fe1 = (feat1, pos1, batch1)

    cidx, cpos = _fps_pallas(pos1, 512)
    cb = batch1[cidx]
    idx, mask = _knn(pos1, cpos, 2.0, 8)
    msg = jnp.concatenate([feat1[idx], pos1[idx] - cpos[:, None, :]], axis=-1)
    feat2 = _pool(msg, mask, params_sc1)
    fe2 = (feat2, cpos, cb)

    cidx2, cpos2 = _fps_pallas(cpos, 128)
    cb2 = cb[cidx2]
    idx, mask = _knn(cpos, cpos2, 4.0, 8)
    msg = jnp.concatenate([feat2[idx], cpos[idx] - cpos2[:, None, :]], axis=-1)
    feat3 = _pool(msg, mask, params_sc2)
    fe3 = (feat3, cpos2, cb2)

    return (fe1, fe2, fe3)


# speedup vs baseline: 2.3548x; 1.3194x over previous
import functools

import jax
import jax.numpy as jnp
import numpy as np
from jax import lax
from jax.experimental import pallas as pl
from jax.experimental.pallas import tpu as pltpu
from jax.experimental.pallas import tpu_sc as plsc

_NC = 2
_NS = 16


def _select_body(S, TPT, K, r2, r2bits_i, active,
                 xs_h, ys_h, zs_h, xt_h, yt_h, zt_h,
                 idx_h, cnt_h,
                 xs_v, ys_v, zs_v, xt_v, yt_v, zt_v, idx_v, idx2_v, cnt_v):
    wid = lax.axis_index("s") * _NC + lax.axis_index("c")
    n_chunks = S // 16
    RS = K + 16

    @pl.when(wid < active)
    def _():
        base = wid * TPT
        pltpu.sync_copy(xs_h, xs_v)
        pltpu.sync_copy(ys_h, ys_v)
        pltpu.sync_copy(zs_h, zs_v)
        pltpu.sync_copy(xt_h.at[pl.ds(base, TPT)], xt_v)
        pltpu.sync_copy(yt_h.at[pl.ds(base, TPT)], yt_v)
        pltpu.sync_copy(zt_h.at[pl.ds(base, TPT)], zt_v)

        lanes = lax.iota(jnp.int32, 16)
        zeros = jnp.zeros((16,), jnp.int32)

        def shift_sum(v):
            for kk in (1, 2, 4, 8):
                sh = v.at[jnp.maximum(lanes - kk, 0)].get(
                    mode="promise_in_bounds")
                v = v + jnp.where(lanes >= kk, sh, 0)
            return v

        def shift_min(v):
            for kk in (1, 2, 4, 8):
                sh = v.at[jnp.maximum(lanes - kk, 0)].get(
                    mode="promise_in_bounds")
                v = jnp.minimum(v, jnp.where(lanes >= kk, sh, v))
            return v

        def d2_chunk(c, tx, ty, tz):
            off = c * 16
            dx = xs_v[pl.ds(off, 16)] - tx
            dy = ys_v[pl.ds(off, 16)] - ty
            dz = zs_v[pl.ds(off, 16)] - tz
            return dx * dx + dy * dy + dz * dz, off

        def per_target(t, carry):
            tg = lax.div(t, jnp.int32(16))
            ti = lax.rem(t, jnp.int32(16))
            tsplat = jnp.full((16,), ti, jnp.int32)
            tx = xt_v[pl.ds(tg * 16, 16)].at[tsplat].get(
                mode="promise_in_bounds")
            ty = yt_v[pl.ds(tg * 16, 16)].at[tsplat].get(
                mode="promise_in_bounds")
            tz = zt_v[pl.ds(tg * 16, 16)].at[tsplat].get(
                mode="promise_in_bounds")
            rbase = t * RS

            def compact_pass(thr_vec):
                def compact(c, cur):
                    d2, off = d2_chunk(c, tx, ty, tz)
                    m = d2 <= thr_vec
                    ranks = shift_sum(jnp.where(m, 1, 0).astype(jnp.int32))
                    nsel = ranks[15]

                    @pl.when(nsel > 0)
                    def _():
                        def extract(p, _):
                            ind = jnp.where(m, ranks, 0) == p + 1
                            liv = shift_min(jnp.where(ind, lanes,
                                                      jnp.int32(16)))
                            j = off + liv[15]
                            slot = cur + p

                            @pl.when(slot < K)
                            def _():
                                idx_v[pl.ds(rbase + slot, 16)] = (
                                    jnp.full((16,), j, jnp.int32))
                            return 0

                        lax.fori_loop(0, nsel, extract, 0)

                    return cur + nsel

                return lax.fori_loop(0, n_chunks, compact, jnp.int32(0))

            tot = compact_pass(jnp.full((16,), jnp.float32(r2)))

            @pl.when(tot > K)
            def _():
                def bstep(_, lh):
                    lo, hi = lh
                    mid = lax.div(lo + hi, jnp.int32(2))
                    tf = jnp.full(
                        (16,), lax.bitcast_convert_type(mid, jnp.float32))

                    def cchunk(c, acc):
                        d2, _ = d2_chunk(c, tx, ty, tz)
                        cs = shift_sum(jnp.where(d2 <= tf, 1, 0)
                                       .astype(jnp.int32))
                        return acc + cs[15]

                    cv = lax.fori_loop(0, n_chunks, cchunk, jnp.int32(0))
                    ge = cv >= K
                    return (jnp.where(ge, lo, mid + 1),
                            jnp.where(ge, mid, hi))

                _, hi = lax.fori_loop(0, 32, bstep,
                                      (jnp.int32(0), jnp.int32(r2bits_i)))
                compact_pass(jnp.full(
                    (16,), lax.bitcast_convert_type(hi, jnp.float32)))

            cnt_v[pl.ds(t, 16)] = jnp.full((16,), jnp.minimum(tot, K),
                                           jnp.int32)
            return carry

        def zchunk(c, carry):
            idx_v[pl.ds(c * 16, 16)] = zeros
            return carry

        lax.fori_loop(0, (TPT * RS) // 16, zchunk, 0)
        lax.fori_loop(0, TPT, per_target, 0)

        def repack(t, carry):
            for q in range(-(-K // 16)):
                v = idx_v[pl.ds(t * RS + q * 16, 16)]
                idx2_v[pl.ds(t * K + q * 16, 16)] = v
            return carry

        lax.fori_loop(0, TPT, repack, 0)

        pltpu.sync_copy(idx2_v.at[pl.ds(0, TPT * K)],
                        idx_h.at[pl.ds(base * K, TPT * K)])
        pltpu.sync_copy(cnt_v.at[pl.ds(0, TPT)],
                        cnt_h.at[pl.ds(base, TPT)])


def _sc_select(pos_src, pos_tgt, r, K, active):
    S = pos_src.shape[0]
    T = pos_tgt.shape[0]
    TPT = T // active
    srcT = pos_src.T
    tgtT = pos_tgt.T
    r2 = float(r) * float(r)
    r2bits_i = int(np.float32(r2).view(np.int32))
    body = functools.partial(_select_body, S, TPT, K, r2, r2bits_i, active)
    mesh = plsc.VectorSubcoreMesh(core_axis_name="c", subcore_axis_name="s")
    f32 = jnp.float32
    idx, cnt = pl.kernel(
        body,
        out_type=(jax.ShapeDtypeStruct((T * K,), jnp.int32),
                  jax.ShapeDtypeStruct((T,), jnp.int32)),
        mesh=mesh,
        scratch_types=[
            pltpu.VMEM((S,), f32), pltpu.VMEM((S,), f32),
            pltpu.VMEM((S,), f32),
            pltpu.VMEM((TPT,), f32), pltpu.VMEM((TPT,), f32),
            pltpu.VMEM((TPT,), f32),
            pltpu.VMEM((TPT * (K + 16),), jnp.int32),
            pltpu.VMEM((TPT * K + 16,), jnp.int32),
            pltpu.VMEM((TPT + 16,), jnp.int32),
        ],
    )(srcT[0], srcT[1], srcT[2], tgtT[0], tgtT[1], tgtT[2])
    return idx.reshape(T, K), cnt


def _fps_body(m, posx_ref, posy_ref, posz_ref, idx_ref, cpos_ref):
    shape = posx_ref.shape
    px = posx_ref[...]
    py = posy_ref[...]
    pz = posz_ref[...]
    row = jax.lax.broadcasted_iota(jnp.int32, shape, 0)
    col = jax.lax.broadcasted_iota(jnp.int32, shape, 1)
    flat = row * 128 + col
    BIG = jnp.int32(2**30)

    def take(v, nxt):
        return (jnp.where(flat == nxt, v, 0.0)).sum()

    def dist_to(nxt):
        x0 = take(px, nxt); y0 = take(py, nxt); z0 = take(pz, nxt)
        return (px - x0) ** 2 + (py - y0) ** 2 + (pz - z0) ** 2, (x0, y0, z0)

    d0, (x0, y0, z0) = dist_to(jnp.int32(0))
    idx_ref[0] = jnp.int32(0)
    cpos_ref[0, 0] = x0
    cpos_ref[0, 1] = y0
    cpos_ref[0, 2] = z0

    def body(i, dists):
        mx = jnp.max(dists)
        nxt = jnp.min(jnp.where(dists == mx, flat, BIG))
        idx_ref[i] = nxt
        d, (x0, y0, z0) = dist_to(nxt)
        cpos_ref[i, 0] = x0
        cpos_ref[i, 1] = y0
        cpos_ref[i, 2] = z0
        return jnp.minimum(dists, d)

    jax.lax.fori_loop(1, m, body, d0)


def _fps_pallas(pos, m):
    n = pos.shape[0]
    posT = pos.T.reshape(3, n // 128, 128)
    idx, cpos = pl.pallas_call(
        functools.partial(_fps_body, m),
        out_shape=(jax.ShapeDtypeStruct((m,), jnp.int32),
                   jax.ShapeDtypeStruct((m, 3), jnp.float32)),
        in_specs=[pl.BlockSpec(memory_space=pltpu.VMEM)] * 3,
        out_specs=(pl.BlockSpec(memory_space=pltpu.SMEM),
                   pl.BlockSpec(memory_space=pltpu.SMEM)),
    )(posT[0], posT[1], posT[2])
    return idx, cpos


def _knn(pos_src, pos_tgt, r, K):
    d2 = jnp.sum((pos_tgt[:, None, :] - pos_src[None, :, :]) ** 2, axis=-1)
    neg, idx = jax.lax.top_k(-d2, K)
    mask = (-neg) <= r * r
    return idx, mask


def _fps(pos, m):
    idxs = jnp.zeros((m,), jnp.int32)
    dists = jnp.sum((pos - pos[0]) ** 2, axis=1)

    def body(i, st):
        dists, idxs = st
        nxt = jnp.argmax(dists).astype(jnp.int32)
        idxs = idxs.at[i].set(nxt)
        d = jnp.sum((pos - pos[nxt]) ** 2, axis=1)
        return (jnp.minimum(dists, d), idxs)

    dists, idxs = jax.lax.fori_loop(1, m, body, (dists, idxs))
    return idxs


def _mlp_bn(x, params, mask):
    m = mask.astype(x.dtype)
    cnt = jnp.maximum(m.sum(), 1.0)
    for (W, b, g, be) in params:
        x = x @ W.T + b
        mean = (x * m[:, None]).sum(axis=0) / cnt
        var = (((x - mean) ** 2) * m[:, None]).sum(axis=0) / cnt
        x = (x - mean) / jnp.sqrt(var + 1e-5) * g + be
        x = jax.nn.relu(x)
    return x


def _pool(msg, mask, params):
    N, K, Cin = msg.shape
    h = _mlp_bn(msg.reshape(N * K, Cin), params, mask.reshape(-1))
    h = h.reshape(N, K, -1)
    h = jnp.where(mask[:, :, None], h, -jnp.inf)
    pooled = h.max(axis=1)
    pooled = jnp.where(mask.any(axis=1)[:, None], pooled, 0.0)
    return pooled


def _copy_kernel(x_ref, o_ref):
    o_ref[...] = x_ref[...]


def _pallas_identity(x):
    return pl.pallas_call(
        _copy_kernel,
        out_shape=jax.ShapeDtypeStruct(x.shape, x.dtype),
    )(x)


def kernel(f1, pos1, batch1, f2, pos2, batch2, params_fe, params_sc1, params_sc2):
    idx, cnt = _sc_select(pos2, pos1, 5.0, 64, 32)
    mask = jnp.arange(64)[None, :] < cnt[:, None]
    N = f1.shape[0]
    msg = jnp.concatenate(
        [jnp.broadcast_to(f1[:, None, :], (N, 64, f1.shape[1])), f2[idx],
         pos2[idx] - pos1[:, None, :]], axis=-1)
    feat1 = _pool(msg, mask, params_fe)
    feat1 = _pallas_identity(feat1)
    fe1 = (feat1, pos1, batch1)

    cidx, cpos = _fps_pallas(pos1, 512)
    cb = batch1[cidx]
    idx, cnt = _sc_select(pos1, cpos, 2.0, 8, 32)
    mask = jnp.arange(8)[None, :] < cnt[:, None]
    msg = jnp.concatenate([feat1[idx], pos1[idx] - cpos[:, None, :]], axis=-1)
    feat2 = _pool(msg, mask, params_sc1)
    fe2 = (feat2, cpos, cb)

    cidx2, cpos2 = _fps_pallas(cpos, 128)
    cb2 = cb[cidx2]
    idx, cnt = _sc_select(cpos, cpos2, 4.0, 8, 8)
    mask = jnp.arange(8)[None, :] < cnt[:, None]
    msg = jnp.concatenate([feat2[idx], cpos[idx] - cpos2[:, None, :]], axis=-1)
    feat3 = _pool(msg, mask, params_sc2)
    fe3 = (feat3, cpos2, cb2)

    return (fe1, fe2, fe3)
